# baseline (speedup 1.0000x reference)
import functools
import jax
import jax.numpy as jnp
from jax.experimental import pallas as pl
from jax.experimental.pallas import tpu as pltpu


def _attn_body(x_ref, y_ref, qw_ref, kvw_ref, o_ref, k_s, v_s, *, heads, dh):
    i = pl.program_id(1)

    @pl.when(i == 0)
    def _():
        yb = y_ref[...]
        c = yb.shape[1]
        kv = jnp.dot(yb, kvw_ref[...], preferred_element_type=jnp.float32)
        k_s[...] = kv[:, :c].astype(jnp.bfloat16)
        v_s[...] = kv[:, c:].astype(jnp.bfloat16)

    q2 = jnp.dot(x_ref[...], qw_ref[...], preferred_element_type=jnp.float32)
    q2 = q2.astype(jnp.bfloat16)
    outs = []
    for hh in range(heads):
        sl = slice(hh * dh, (hh + 1) * dh)
        s = jax.lax.dot_general(q2[:, sl], k_s[:, sl], (((1,), (1,)), ((), ())),
                                preferred_element_type=jnp.float32)
        sb = s.astype(jnp.bfloat16)
        m = jnp.max(sb, axis=-1, keepdims=True)
        p = jnp.exp(sb - m)
        denom = jnp.sum(p.astype(jnp.float32), axis=-1, keepdims=True)
        o = jnp.dot(p, v_s[:, sl], preferred_element_type=jnp.float32)
        outs.append(o / denom)
    o_ref[...] = jnp.concatenate(outs, axis=1).astype(o_ref.dtype)


def _gelu_exact(h):
    return 0.5 * h * (1.0 + jax.lax.erf(h * (2.0 ** -0.5)))


def _moe_body(bnd_ref, z_ref, sw1_ref, sb1_ref, sw2_ref, sb2_ref,
              lw1_ref, lb1_ref, lw2_ref, lb2_ref, o_ref, *, tm):
    t = pl.program_id(0)
    bnd = bnd_ref[t]
    zb = z_ref[...]

    def expert(w1_ref, b1_ref, w2_ref, b2_ref):
        h = jnp.dot(zb, w1_ref[...], preferred_element_type=jnp.float32)
        h = _gelu_exact(h + b1_ref[...]).astype(jnp.bfloat16)
        return jnp.dot(h, w2_ref[...], preferred_element_type=jnp.float32
                       ) + b2_ref[...]

    @pl.when(bnd == tm)
    def _():
        o_ref[...] = expert(sw1_ref, sb1_ref, sw2_ref, sb2_ref)

    @pl.when(bnd == 0)
    def _():
        o_ref[...] = expert(lw1_ref, lb1_ref, lw2_ref, lb2_ref)

    @pl.when(jnp.logical_and(bnd > 0, bnd < tm))
    def _():
        o_s = expert(sw1_ref, sb1_ref, sw2_ref, sb2_ref)
        o_l = expert(lw1_ref, lb1_ref, lw2_ref, lb2_ref)
        rows = jax.lax.broadcasted_iota(jnp.int32, o_s.shape, 0)
        o_ref[...] = jnp.where(rows < bnd, o_s, o_l)


def kernel(x, y, token_types, q_w, kv_w, s_w1, s_b1, s_w2, s_b2,
           l_w1, l_b1, l_w2, l_b2):
    b, n, c = x.shape
    heads = 12
    dh = c // heads
    hid = s_w1.shape[1]
    scale = dh ** -0.5
    tm = 512
    nt = n // tm
    bn = b * n

    xf = x.reshape(bn, c).astype(jnp.bfloat16)
    yf = y.reshape(bn, c).astype(jnp.bfloat16)
    qws = (q_w * scale).astype(jnp.bfloat16)
    kvwb = kv_w.astype(jnp.bfloat16)

    attn_out = pl.pallas_call(
        functools.partial(_attn_body, heads=heads, dh=dh),
        grid=(b, nt),
        in_specs=[
            pl.BlockSpec((tm, c), lambda bi, i: (bi * (n // tm) + i, 0)),
            pl.BlockSpec((n, c), lambda bi, i: (bi, 0)),
            pl.BlockSpec((c, c), lambda bi, i: (0, 0)),
            pl.BlockSpec((c, 2 * c), lambda bi, i: (0, 0)),
        ],
        out_specs=pl.BlockSpec((tm, c), lambda bi, i: (bi * (n // tm) + i, 0)),
        out_shape=jax.ShapeDtypeStruct((bn, c), jnp.bfloat16),
        scratch_shapes=[
            pltpu.VMEM((n, c), jnp.bfloat16),
            pltpu.VMEM((n, c), jnp.bfloat16),
        ],
        compiler_params=pltpu.CompilerParams(
            dimension_semantics=("parallel", "arbitrary")),
    )(xf, yf, qws, kvwb)

    tm2 = 128
    tpr = n // tm2
    nt2 = bn // tm2
    cnt = jnp.sum((token_types == 0).astype(jnp.int32), axis=-1)
    tidx = jnp.arange(nt2, dtype=jnp.int32)
    lo = (tidx % tpr) * tm2
    bnd = jnp.clip(cnt[tidx // tpr] - lo, 0, tm2).astype(jnp.int32)

    wspec = pl.BlockSpec((c, hid), lambda t: (0, 0))
    w2spec = pl.BlockSpec((hid, c), lambda t: (0, 0))
    b1spec = pl.BlockSpec((1, hid), lambda t: (0, 0))
    b2spec = pl.BlockSpec((1, c), lambda t: (0, 0))

    out = pl.pallas_call(
        functools.partial(_moe_body, tm=tm2),
        grid=(nt2,),
        in_specs=[
            pl.BlockSpec(memory_space=pltpu.SMEM),
            pl.BlockSpec((tm2, c), lambda t: (t, 0)),
            wspec, b1spec, w2spec, b2spec,
            wspec, b1spec, w2spec, b2spec,
        ],
        out_specs=pl.BlockSpec((tm2, c), lambda t: (t, 0)),
        out_shape=jax.ShapeDtypeStruct((bn, c), jnp.float32),
        compiler_params=pltpu.CompilerParams(
            dimension_semantics=("parallel",)),
    )(bnd, attn_out,
      s_w1.astype(jnp.bfloat16), s_b1.reshape(1, hid),
      s_w2.astype(jnp.bfloat16), s_b2.reshape(1, c),
      l_w1.astype(jnp.bfloat16), l_b1.reshape(1, hid),
      l_w2.astype(jnp.bfloat16), l_b2.reshape(1, c))

    return out.reshape(b, n, c)


# speedup vs baseline: 1.1730x; 1.1730x over previous
import functools
import jax
import jax.numpy as jnp
from jax.experimental import pallas as pl
from jax.experimental.pallas import tpu as pltpu


def _attn_body(x_ref, y_ref, qw_ref, kvw_ref,
               sw1f_ref, sw2f_ref, lw1f_ref, lw2f_ref,
               o_ref, sw1_ref, sw2_ref, lw1_ref, lw2_ref,
               k_s, v_s, *, heads, dh):
    i = pl.program_id(1)

    @pl.when(i == 0)
    def _():
        yb = y_ref[...].astype(jnp.bfloat16)
        c = yb.shape[1]
        nk = yb.shape[0]
        kv = jnp.dot(yb, kvw_ref[...].astype(jnp.bfloat16),
                     preferred_element_type=jnp.float32)
        k_s[...] = kv[:, :c].astype(jnp.bfloat16)
        pad = jnp.concatenate(
            [jnp.ones((nk, 1), jnp.bfloat16),
             jnp.zeros((nk, 2 * dh - dh - 1), jnp.bfloat16)], axis=1)
        vparts = []
        for hh in range(heads):
            vparts.append(kv[:, c + hh * dh:c + (hh + 1) * dh
                             ].astype(jnp.bfloat16))
            vparts.append(pad)
        v_s[...] = jnp.concatenate(vparts, axis=1)

    q2 = jnp.dot(x_ref[...].astype(jnp.bfloat16), qw_ref[...],
                 preferred_element_type=jnp.float32)
    q2 = q2.astype(jnp.bfloat16)
    outs = []
    for hh in range(heads):
        sl = slice(hh * dh, (hh + 1) * dh)
        sb = jax.lax.dot_general(q2[:, sl], k_s[:, sl],
                                 (((1,), (1,)), ((), ())),
                                 preferred_element_type=jnp.float32
                                 ).astype(jnp.bfloat16)
        m = jnp.max(sb, axis=-1, keepdims=True)
        p = jnp.exp(sb - m)
        oa = jnp.dot(p, v_s[:, hh * 2 * dh:(hh + 1) * 2 * dh],
                     preferred_element_type=jnp.float32)
        outs.append(oa[:, :dh] / oa[:, dh:dh + 1])
    o_ref[...] = jnp.concatenate(outs, axis=1).astype(o_ref.dtype)
    sw1_ref[...] = sw1f_ref[...].astype(jnp.bfloat16)
    sw2_ref[...] = sw2f_ref[...].astype(jnp.bfloat16)
    lw1_ref[...] = lw1f_ref[...].astype(jnp.bfloat16)
    lw2_ref[...] = lw2f_ref[...].astype(jnp.bfloat16)


def _gelu_exact(h):
    return 0.5 * h * (1.0 + jax.lax.erf(h * (2.0 ** -0.5)))


def _moe_body(bnd_ref, z_ref, sw1_ref, sb1_ref, sw2_ref, sb2_ref,
              lw1_ref, lb1_ref, lw2_ref, lb2_ref, o_ref, *, tm):
    t = pl.program_id(0)
    bnd = bnd_ref[t]
    zb = z_ref[...]

    def expert(w1_ref, b1_ref, w2_ref, b2_ref):
        h = jnp.dot(zb, w1_ref[...], preferred_element_type=jnp.float32)
        h = _gelu_exact(h + b1_ref[...]).astype(jnp.bfloat16)
        return jnp.dot(h, w2_ref[...], preferred_element_type=jnp.float32
                       ) + b2_ref[...]

    @pl.when(bnd == tm)
    def _():
        o_ref[...] = expert(sw1_ref, sb1_ref, sw2_ref, sb2_ref)

    @pl.when(bnd == 0)
    def _():
        o_ref[...] = expert(lw1_ref, lb1_ref, lw2_ref, lb2_ref)

    @pl.when(jnp.logical_and(bnd > 0, bnd < tm))
    def _():
        o_s = expert(sw1_ref, sb1_ref, sw2_ref, sb2_ref)
        o_l = expert(lw1_ref, lb1_ref, lw2_ref, lb2_ref)
        rows = jax.lax.broadcasted_iota(jnp.int32, o_s.shape, 0)
        o_ref[...] = jnp.where(rows < bnd, o_s, o_l)


def kernel(x, y, token_types, q_w, kv_w, s_w1, s_b1, s_w2, s_b2,
           l_w1, l_b1, l_w2, l_b2):
    b, n, c = x.shape
    heads = 12
    dh = c // heads
    hid = s_w1.shape[1]
    scale = dh ** -0.5
    tm = 512
    nt = n // tm
    bn = b * n

    xf = x.reshape(bn, c)
    yf = y.reshape(bn, c)
    qws = (q_w * scale).astype(jnp.bfloat16)
    kvwb = kv_w

    steps = b * nt
    c1 = c // steps
    c2 = hid // steps
    w1cs = pl.BlockSpec((c1, hid), lambda bi, i, _nt=nt: (bi * _nt + i, 0))
    w2cs = pl.BlockSpec((c2, c), lambda bi, i, _nt=nt: (bi * _nt + i, 0))
    attn_out, sw1b, sw2b, lw1b, lw2b = pl.pallas_call(
        functools.partial(_attn_body, heads=heads, dh=dh),
        grid=(b, nt),
        in_specs=[
            pl.BlockSpec((tm, c), lambda bi, i: (bi * (n // tm) + i, 0)),
            pl.BlockSpec((n, c), lambda bi, i: (bi, 0)),
            pl.BlockSpec((c, c), lambda bi, i: (0, 0)),
            pl.BlockSpec((c, 2 * c), lambda bi, i: (0, 0)),
            w1cs, w2cs, w1cs, w2cs,
        ],
        out_specs=(
            pl.BlockSpec((tm, c), lambda bi, i: (bi * (n // tm) + i, 0)),
            w1cs, w2cs, w1cs, w2cs,
        ),
        out_shape=(
            jax.ShapeDtypeStruct((bn, c), jnp.bfloat16),
            jax.ShapeDtypeStruct((c, hid), jnp.bfloat16),
            jax.ShapeDtypeStruct((hid, c), jnp.bfloat16),
            jax.ShapeDtypeStruct((c, hid), jnp.bfloat16),
            jax.ShapeDtypeStruct((hid, c), jnp.bfloat16),
        ),
        scratch_shapes=[
            pltpu.VMEM((n, c), jnp.bfloat16),
            pltpu.VMEM((n, 2 * c), jnp.bfloat16),
        ],
        compiler_params=pltpu.CompilerParams(
            dimension_semantics=("parallel", "arbitrary")),
    )(xf, yf, qws, kvwb, s_w1, s_w2, l_w1, l_w2)

    tm2 = 256
    tpr = n // tm2
    nt2 = bn // tm2
    cnt = jnp.sum((token_types == 0).astype(jnp.int32), axis=-1)
    tidx = jnp.arange(nt2, dtype=jnp.int32)
    lo = (tidx % tpr) * tm2
    bnd = jnp.clip(cnt[tidx // tpr] - lo, 0, tm2).astype(jnp.int32)

    wspec = pl.BlockSpec((c, hid), lambda t: (0, 0))
    w2spec = pl.BlockSpec((hid, c), lambda t: (0, 0))
    b1spec = pl.BlockSpec((1, hid), lambda t: (0, 0))
    b2spec = pl.BlockSpec((1, c), lambda t: (0, 0))

    out = pl.pallas_call(
        functools.partial(_moe_body, tm=tm2),
        grid=(nt2,),
        in_specs=[
            pl.BlockSpec(memory_space=pltpu.SMEM),
            pl.BlockSpec((tm2, c), lambda t: (t, 0)),
            wspec, b1spec, w2spec, b2spec,
            wspec, b1spec, w2spec, b2spec,
        ],
        out_specs=pl.BlockSpec((tm2, c), lambda t: (t, 0)),
        out_shape=jax.ShapeDtypeStruct((bn, c), jnp.float32),
        compiler_params=pltpu.CompilerParams(
            dimension_semantics=("parallel",)),
    )(bnd, attn_out,
      sw1b, s_b1.reshape(1, hid), sw2b, s_b2.reshape(1, c),
      lw1b, l_b1.reshape(1, hid), lw2b, l_b2.reshape(1, c))

    return out.reshape(b, n, c)
